# baseline (speedup 1.0000x reference)
import functools
import math

import jax
import jax.numpy as jnp
from jax import lax
from jax.experimental import pallas as pl
from jax.experimental.pallas import tpu as pltpu
from jax.experimental.pallas import tpu_sc as plsc

PI_F = float(math.pi)

_NC = 2
_NS = 16
_L = 16
_NW = _NC * _NS


def _row_to_channel(r):
    return 2 * r if r < 8 else 2 * (r - 8) + 1


@functools.lru_cache(maxsize=None)
def _make_sc_gather(B, C, H, W, K, half):
    P = B * K
    PH = P // 2
    assert P % _NW == 0
    n = P // _NW
    assert n % _L == 0
    assert K % n == 0
    assert C == 16
    assert W == 256
    _NBUF = 4

    mesh = plsc.VectorSubcoreMesh(
        core_axis_name="c", subcore_axis_name="s", num_cores=1)

    @functools.partial(
        pl.kernel,
        mesh=mesh,
        compiler_params=pltpu.CompilerParams(needs_layout_passes=False),
        out_type=(
            jax.ShapeDtypeStruct((16 * PH,), jnp.float32),
            jax.ShapeDtypeStruct((16 * PH,), jnp.float32),
        ),
        scratch_types=[
            pltpu.VMEM((n,), jnp.int32),
            pltpu.VMEM((16, n), jnp.int32),
            pltpu.VMEM((16, n), jnp.int32),
            pltpu.VMEM((16, n), jnp.float32),
            pltpu.VMEM((16, n), jnp.float32),
            pltpu.VMEM((_NBUF, n, W), jnp.float32),
            pltpu.SemaphoreType.DMA,
            pltpu.SemaphoreType.DMA,
            pltpu.SemaphoreType.DMA,
            pltpu.SemaphoreType.DMA,
            pltpu.SemaphoreType.DMA,
        ],
    )
    def sc_gather(outv, tarf, indf, pred_t, tar_t,
                  ind_v, idxp, idxt, gp, gt, rbuf,
                  semt, s0, s1, s2, s3):
        sems = [s0, s1, s2, s3]
        wid = lax.axis_index("s")
        lbase = wid * n
        base = half * PH + lbase
        pltpu.sync_copy(indf.at[pl.ds(base, n)], ind_v)
        b_s = base // K
        for t in range(n // _L):
            iv = ind_v[pl.ds(t * _L, _L)]
            hv = jnp.right_shift(iv, 8)
            pv = base + t * _L + lax.iota(jnp.int32, _L)
            for r in range(16):
                c = _row_to_channel(r)
                idxp[r, pl.ds(t * _L, _L)] = hv + (b_s * C + c) * H
                idxt[r, pl.ds(t * _L, _L)] = pv * C + c
        tcopies = []
        for r in range(16):
            tcopies.append(
                pltpu.async_copy(tarf.at[idxt.at[r]], gt.at[r], semt))
        copies = [None] * 16
        for r in range(_NBUF):
            copies[r] = pltpu.async_copy(
                outv.at[idxp.at[r]], rbuf.at[r], sems[r])
        for r in range(16):
            copies[r].wait()
            rb = rbuf.at[r % _NBUF]
            for t in range(n // _L):
                iv = ind_v[pl.ds(t * _L, _L)]
                wv = jnp.bitwise_and(iv, W - 1)
                ivec = t * _L + lax.iota(jnp.int32, _L)
                vals = plsc.load_gather(rb, [ivec, wv])
                gp[r, pl.ds(t * _L, _L)] = vals
            if r + _NBUF < 16:
                copies[r + _NBUF] = pltpu.async_copy(
                    outv.at[idxp.at[r + _NBUF]],
                    rbuf.at[(r + _NBUF) % _NBUF],
                    sems[(r + _NBUF) % _NBUF])
        for cp in tcopies:
            cp.wait()
        for r in range(16):
            pltpu.sync_copy(gp.at[r], pred_t.at[pl.ds(r * PH + lbase, n)])
            pltpu.sync_copy(gt.at[r], tar_t.at[pl.ds(r * PH + lbase, n)])

    return sc_gather


def _tc_body(pred0_ref, pred1_ref, tart0_ref, tart1_ref, mask_ref, out_ref):
    NB = mask_ref.shape[0]
    NH = NB // 2
    m = mask_ref[...][None]
    pred = jnp.concatenate(
        [pred0_ref[...].reshape(16, NH, 128),
         pred1_ref[...].reshape(16, NH, 128)], axis=1)
    tart = jnp.concatenate(
        [tart0_ref[...].reshape(16, NH, 128),
         tart1_ref[...].reshape(16, NH, 128)], axis=1)
    dp = pred[0:8] * m
    ap = pred[8:16] * m
    dt = tart[0:8] * m
    at = tart[8:16] * m

    rad = PI_F / 180.0
    px = dp * jnp.cos(ap * rad)
    py = dp * jnp.sin(ap * rad)
    tx = dt * jnp.cos(at * rad)
    ty = dt * jnp.sin(at * rad)

    def roll_prev(a):
        return jnp.concatenate([a[7:8], a[0:7]], axis=0)

    def roll_next(a):
        return jnp.concatenate([a[1:8], a[0:1]], axis=0)

    pvx, pvy = roll_prev(tx), roll_prev(ty)
    nvx, nvy = roll_next(tx), roll_next(ty)

    def closest(ax, ay, bx, by):
        abx = bx - ax
        aby = by - ay
        t = ((px - ax) * abx + (py - ay) * aby) / (abx * abx + aby * aby)
        t = jnp.clip(t, 0.0, 1.0)
        cx = ax + t * abx
        cy = ay + t * aby
        d = jnp.sqrt((px - cx) ** 2 + (py - cy) ** 2)
        return cx, cy, d

    c1x, c1y, d1 = closest(pvx, pvy, tx, ty)
    c2x, c2y, d2 = closest(tx, ty, nvx, nvy)
    use2 = d2 < d1
    chx = jnp.where(use2, c2x, c1x)
    chy = jnp.where(use2, c2y, c1y)

    ri = lax.broadcasted_iota(jnp.int32, (8, NB, 128), 0)
    is_true_kp = (ri == 0) | (ri == 3) | (ri == 7)
    same_nb = (pvx == nvx) & (pvy == nvy)
    use_t = is_true_kp | same_nb
    prx = jnp.where(use_t, tx, chx)
    pry = jnp.where(use_t, ty, chy)

    keep = jnp.broadcast_to(m != 0.0, (8, NB, 128))
    prx = jnp.where(keep, prx, 0.0)
    pry = jnp.where(keep, pry, 0.0)

    dist = jnp.sqrt(prx * prx + pry * pry)
    ang = jnp.arctan2(pry, prx) * (180.0 / PI_F)
    ang = jnp.where(ang < 0.0, ang + 360.0, ang)

    total = jnp.sum(jnp.abs(dp * m - dist * m)) + \
        jnp.sum(jnp.abs(ap * m - ang * m))
    denom = jnp.sum(m) * 16.0 + 0.0001
    out_ref[...] = (total / denom).reshape(1, 1)


def kernel(output, mask, ind, target):
    B, C, H, W = output.shape
    K = ind.shape[1]
    P = B * K
    outv = output.reshape(B * C * H, W)
    tarf = target.reshape(P * C)
    indf = ind.reshape(P).astype(jnp.int32)
    p0, t0 = _make_sc_gather(B, C, H, W, K, 0)(outv, tarf, indf)
    p1, t1 = _make_sc_gather(B, C, H, W, K, 1)(outv, tarf, indf)
    PH = P // 2
    loss = pl.pallas_call(
        _tc_body,
        out_shape=jax.ShapeDtypeStruct((1, 1), jnp.float32),
    )(p0.reshape(16 * PH // 128, 128), p1.reshape(16 * PH // 128, 128),
      t0.reshape(16 * PH // 128, 128), t1.reshape(16 * PH // 128, 128),
      mask.reshape(P // 128, 128))
    return loss[0, 0]


# speedup vs baseline: 1.4719x; 1.4719x over previous
import functools
import math

import jax
import jax.numpy as jnp
from jax import lax
from jax.experimental import pallas as pl
from jax.experimental.pallas import tpu as pltpu
from jax.experimental.pallas import tpu_sc as plsc

PI_F = float(math.pi)

_NC = 2
_NS = 16
_L = 16
_NW = _NC * _NS


def _row_to_channel(r):
    return 2 * r if r < 8 else 2 * (r - 8) + 1


@functools.lru_cache(maxsize=None)
def _make_sc_gather(B, C, H, W, K):
    P = B * K
    assert P % _NW == 0
    n = P // _NW
    assert n % _L == 0
    assert K % n == 0
    assert C == 16
    assert W == 256
    _NBUF = 4

    mesh = plsc.VectorSubcoreMesh(core_axis_name="c", subcore_axis_name="s")

    @functools.partial(
        pl.kernel,
        mesh=mesh,
        compiler_params=pltpu.CompilerParams(needs_layout_passes=False),
        out_type=(
            jax.ShapeDtypeStruct((16 * P,), jnp.float32),
            jax.ShapeDtypeStruct((16 * P,), jnp.float32),
        ),
        scratch_types=[
            pltpu.VMEM((n,), jnp.int32),
            pltpu.VMEM((16, n), jnp.int32),
            pltpu.VMEM((16, n), jnp.int32),
            pltpu.VMEM((16, n), jnp.float32),
            pltpu.VMEM((16, n), jnp.float32),
            pltpu.VMEM((_NBUF, n, W), jnp.float32),
            pltpu.SemaphoreType.DMA,
            pltpu.SemaphoreType.DMA,
            pltpu.SemaphoreType.DMA,
            pltpu.SemaphoreType.DMA,
            pltpu.SemaphoreType.DMA,
        ],
    )
    def sc_gather(outv, tarf, indf, pred_t, tar_t,
                  ind_v, idxp, idxt, gp, gt, rbuf,
                  semt, s0, s1, s2, s3):
        sems = [s0, s1, s2, s3]
        wid = lax.axis_index("s") * _NC + lax.axis_index("c")
        base = wid * n
        pltpu.sync_copy(indf.at[pl.ds(base, n)], ind_v)
        b_s = base // K
        for t in range(n // _L):
            iv = ind_v[pl.ds(t * _L, _L)]
            hv = jnp.right_shift(iv, 8)
            pv = base + t * _L + lax.iota(jnp.int32, _L)
            for r in range(16):
                c = _row_to_channel(r)
                idxp[r, pl.ds(t * _L, _L)] = hv + (b_s * C + c) * H
                idxt[r, pl.ds(t * _L, _L)] = pv * C + c
        tcopies = []
        for r in range(16):
            tcopies.append(
                pltpu.async_copy(tarf.at[idxt.at[r]], gt.at[r], semt))
        copies = [None] * 16
        for r in range(_NBUF):
            copies[r] = pltpu.async_copy(
                outv.at[idxp.at[r]], rbuf.at[r], sems[r])
        for r in range(16):
            copies[r].wait()
            rb = rbuf.at[r % _NBUF]
            for t in range(n // _L):
                iv = ind_v[pl.ds(t * _L, _L)]
                wv = jnp.bitwise_and(iv, W - 1)
                ivec = t * _L + lax.iota(jnp.int32, _L)
                vals = plsc.load_gather(rb, [ivec, wv])
                gp[r, pl.ds(t * _L, _L)] = vals
            if r + _NBUF < 16:
                copies[r + _NBUF] = pltpu.async_copy(
                    outv.at[idxp.at[r + _NBUF]],
                    rbuf.at[(r + _NBUF) % _NBUF],
                    sems[(r + _NBUF) % _NBUF])
        for cp in tcopies:
            cp.wait()
        for r in range(16):
            pltpu.sync_copy(gp.at[r], pred_t.at[pl.ds(r * P + base, n)])
            pltpu.sync_copy(gt.at[r], tar_t.at[pl.ds(r * P + base, n)])

    return sc_gather


def _tc_body(pred_ref, tart_ref, mask_ref, out_ref):
    NB = mask_ref.shape[0]
    m = mask_ref[...][None]
    pred = pred_ref[...].reshape(16, NB, 128)
    tart = tart_ref[...].reshape(16, NB, 128)
    dp = pred[0:8] * m
    ap = pred[8:16] * m
    dt = tart[0:8] * m
    at = tart[8:16] * m

    rad = PI_F / 180.0
    px = dp * jnp.cos(ap * rad)
    py = dp * jnp.sin(ap * rad)
    tx = dt * jnp.cos(at * rad)
    ty = dt * jnp.sin(at * rad)

    def roll_prev(a):
        return jnp.concatenate([a[7:8], a[0:7]], axis=0)

    def roll_next(a):
        return jnp.concatenate([a[1:8], a[0:1]], axis=0)

    pvx, pvy = roll_prev(tx), roll_prev(ty)
    nvx, nvy = roll_next(tx), roll_next(ty)

    def closest(ax, ay, bx, by):
        abx = bx - ax
        aby = by - ay
        t = ((px - ax) * abx + (py - ay) * aby) / (abx * abx + aby * aby)
        t = jnp.clip(t, 0.0, 1.0)
        cx = ax + t * abx
        cy = ay + t * aby
        d = jnp.sqrt((px - cx) ** 2 + (py - cy) ** 2)
        return cx, cy, d

    c1x, c1y, d1 = closest(pvx, pvy, tx, ty)
    c2x, c2y, d2 = closest(tx, ty, nvx, nvy)
    use2 = d2 < d1
    chx = jnp.where(use2, c2x, c1x)
    chy = jnp.where(use2, c2y, c1y)

    ri = lax.broadcasted_iota(jnp.int32, (8, NB, 128), 0)
    is_true_kp = (ri == 0) | (ri == 3) | (ri == 7)
    same_nb = (pvx == nvx) & (pvy == nvy)
    use_t = is_true_kp | same_nb
    prx = jnp.where(use_t, tx, chx)
    pry = jnp.where(use_t, ty, chy)

    keep = jnp.broadcast_to(m != 0.0, (8, NB, 128))
    prx = jnp.where(keep, prx, 0.0)
    pry = jnp.where(keep, pry, 0.0)

    dist = jnp.sqrt(prx * prx + pry * pry)
    ang = jnp.arctan2(pry, prx) * (180.0 / PI_F)
    ang = jnp.where(ang < 0.0, ang + 360.0, ang)

    total = jnp.sum(jnp.abs(dp * m - dist * m)) + \
        jnp.sum(jnp.abs(ap * m - ang * m))
    denom = jnp.sum(m) * 16.0 + 0.0001
    out_ref[...] = (total / denom).reshape(1, 1)


def kernel(output, mask, ind, target):
    B, C, H, W = output.shape
    K = ind.shape[1]
    P = B * K
    outv = output.reshape(B * C * H, W)
    tarf = target.reshape(P * C)
    indf = ind.reshape(P).astype(jnp.int32)
    pred_t, tar_t = _make_sc_gather(B, C, H, W, K)(outv, tarf, indf)
    pred_t = pred_t.reshape(16 * P // 128, 128)
    tar_t = tar_t.reshape(16 * P // 128, 128)
    loss = pl.pallas_call(
        _tc_body,
        out_shape=jax.ShapeDtypeStruct((1, 1), jnp.float32),
    )(pred_t, tar_t, mask.reshape(P // 128, 128))
    return loss[0, 0]


# speedup vs baseline: 1.4764x; 1.0031x over previous
import functools
import math

import jax
import jax.numpy as jnp
from jax import lax
from jax.experimental import pallas as pl
from jax.experimental.pallas import tpu as pltpu
from jax.experimental.pallas import tpu_sc as plsc

PI_F = float(math.pi)

_NC = 2
_NS = 16
_L = 16
_NW = _NC * _NS


def _row_to_channel(r):
    return 2 * r if r < 8 else 2 * (r - 8) + 1


@functools.lru_cache(maxsize=None)
def _make_sc_gather(B, C, H, W, K):
    P = B * K
    assert P % _NW == 0
    n = P // _NW
    assert n % _L == 0
    assert K % n == 0
    assert C == 16
    assert W == 256
    _NBUF = 6

    mesh = plsc.VectorSubcoreMesh(core_axis_name="c", subcore_axis_name="s")

    @functools.partial(
        pl.kernel,
        mesh=mesh,
        compiler_params=pltpu.CompilerParams(needs_layout_passes=False),
        out_type=(
            jax.ShapeDtypeStruct((16 * P,), jnp.float32),
            jax.ShapeDtypeStruct((16 * P,), jnp.float32),
        ),
        scratch_types=[
            pltpu.VMEM((n,), jnp.int32),
            pltpu.VMEM((16, n), jnp.int32),
            pltpu.VMEM((16, n), jnp.int32),
            pltpu.VMEM((16, n), jnp.float32),
            pltpu.VMEM((16, n), jnp.float32),
            pltpu.VMEM((_NBUF, n, W), jnp.float32),
            pltpu.SemaphoreType.DMA,
            pltpu.SemaphoreType.DMA,
            pltpu.SemaphoreType.DMA,
            pltpu.SemaphoreType.DMA,
            pltpu.SemaphoreType.DMA,
            pltpu.SemaphoreType.DMA,
            pltpu.SemaphoreType.DMA,
        ],
    )
    def sc_gather(outv, tarf, indf, pred_t, tar_t,
                  ind_v, idxp, idxt, gp, gt, rbuf,
                  semt, s0, s1, s2, s3, s4, s5):
        sems = [s0, s1, s2, s3, s4, s5]
        wid = lax.axis_index("s") * _NC + lax.axis_index("c")
        base = wid * n
        pltpu.sync_copy(indf.at[pl.ds(base, n)], ind_v)
        b_s = base // K
        for t in range(n // _L):
            iv = ind_v[pl.ds(t * _L, _L)]
            hv = jnp.right_shift(iv, 8)
            pv = base + t * _L + lax.iota(jnp.int32, _L)
            for r in range(16):
                c = _row_to_channel(r)
                idxp[r, pl.ds(t * _L, _L)] = hv + (b_s * C + c) * H
                idxt[r, pl.ds(t * _L, _L)] = pv * C + c
        tcopies = []
        for r in range(16):
            tcopies.append(
                pltpu.async_copy(tarf.at[idxt.at[r]], gt.at[r], semt))
        copies = [None] * 16
        for r in range(_NBUF):
            copies[r] = pltpu.async_copy(
                outv.at[idxp.at[r]], rbuf.at[r], sems[r])
        for r in range(16):
            copies[r].wait()
            rb = rbuf.at[r % _NBUF]
            for t in range(n // _L):
                iv = ind_v[pl.ds(t * _L, _L)]
                wv = jnp.bitwise_and(iv, W - 1)
                ivec = t * _L + lax.iota(jnp.int32, _L)
                vals = plsc.load_gather(rb, [ivec, wv])
                gp[r, pl.ds(t * _L, _L)] = vals
            if r + _NBUF < 16:
                copies[r + _NBUF] = pltpu.async_copy(
                    outv.at[idxp.at[r + _NBUF]],
                    rbuf.at[(r + _NBUF) % _NBUF],
                    sems[(r + _NBUF) % _NBUF])
        for cp in tcopies:
            cp.wait()
        for r in range(16):
            pltpu.sync_copy(gp.at[r], pred_t.at[pl.ds(r * P + base, n)])
            pltpu.sync_copy(gt.at[r], tar_t.at[pl.ds(r * P + base, n)])

    return sc_gather


def _tc_body(pred_ref, tart_ref, mask_ref, out_ref):
    NB = mask_ref.shape[0]
    m = mask_ref[...][None]
    pred = pred_ref[...].reshape(16, NB, 128)
    tart = tart_ref[...].reshape(16, NB, 128)
    dp = pred[0:8] * m
    ap = pred[8:16] * m
    dt = tart[0:8] * m
    at = tart[8:16] * m

    rad = PI_F / 180.0
    px = dp * jnp.cos(ap * rad)
    py = dp * jnp.sin(ap * rad)
    tx = dt * jnp.cos(at * rad)
    ty = dt * jnp.sin(at * rad)

    def roll_prev(a):
        return jnp.concatenate([a[7:8], a[0:7]], axis=0)

    def roll_next(a):
        return jnp.concatenate([a[1:8], a[0:1]], axis=0)

    pvx, pvy = roll_prev(tx), roll_prev(ty)
    nvx, nvy = roll_next(tx), roll_next(ty)

    def closest(ax, ay, bx, by):
        abx = bx - ax
        aby = by - ay
        t = ((px - ax) * abx + (py - ay) * aby) / (abx * abx + aby * aby)
        t = jnp.clip(t, 0.0, 1.0)
        cx = ax + t * abx
        cy = ay + t * aby
        d = jnp.sqrt((px - cx) ** 2 + (py - cy) ** 2)
        return cx, cy, d

    c1x, c1y, d1 = closest(pvx, pvy, tx, ty)
    c2x, c2y, d2 = closest(tx, ty, nvx, nvy)
    use2 = d2 < d1
    chx = jnp.where(use2, c2x, c1x)
    chy = jnp.where(use2, c2y, c1y)

    ri = lax.broadcasted_iota(jnp.int32, (8, NB, 128), 0)
    is_true_kp = (ri == 0) | (ri == 3) | (ri == 7)
    same_nb = (pvx == nvx) & (pvy == nvy)
    use_t = is_true_kp | same_nb
    prx = jnp.where(use_t, tx, chx)
    pry = jnp.where(use_t, ty, chy)

    keep = jnp.broadcast_to(m != 0.0, (8, NB, 128))
    prx = jnp.where(keep, prx, 0.0)
    pry = jnp.where(keep, pry, 0.0)

    dist = jnp.sqrt(prx * prx + pry * pry)
    ang = jnp.arctan2(pry, prx) * (180.0 / PI_F)
    ang = jnp.where(ang < 0.0, ang + 360.0, ang)

    total = jnp.sum(jnp.abs(dp * m - dist * m)) + \
        jnp.sum(jnp.abs(ap * m - ang * m))
    denom = jnp.sum(m) * 16.0 + 0.0001
    out_ref[...] = (total / denom).reshape(1, 1)


def kernel(output, mask, ind, target):
    B, C, H, W = output.shape
    K = ind.shape[1]
    P = B * K
    outv = output.reshape(B * C * H, W)
    tarf = target.reshape(P * C)
    indf = ind.reshape(P).astype(jnp.int32)
    pred_t, tar_t = _make_sc_gather(B, C, H, W, K)(outv, tarf, indf)
    pred_t = pred_t.reshape(16 * P // 128, 128)
    tar_t = tar_t.reshape(16 * P // 128, 128)
    loss = pl.pallas_call(
        _tc_body,
        out_shape=jax.ShapeDtypeStruct((1, 1), jnp.float32),
    )(pred_t, tar_t, mask.reshape(P // 128, 128))
    return loss[0, 0]
